# baseline (speedup 1.0000x reference)
import functools

import jax
import jax.numpy as jnp
from jax import lax
from jax.experimental import pallas as pl
from jax.experimental.pallas import tpu as pltpu
from jax.experimental.pallas import tpu_sc as plsc

NC = 2
NS = 16
NW = NC * NS
NBUF = 4
NSPL = 4


@jax.jit
def _sc_gather(idx, table):
    K, B = idx.shape
    _, D = table.shape
    CB = B // NW
    mesh = plsc.VectorSubcoreMesh(core_axis_name="c", subcore_axis_name="s")

    @functools.partial(
        pl.kernel,
        out_type=jax.ShapeDtypeStruct((K, B, D), jnp.float32),
        mesh=mesh,
        compiler_params=pltpu.CompilerParams(use_tc_tiling_on_sc=True),
        scratch_types=[
            pltpu.VMEM((K, CB), jnp.int32),
            [pltpu.VMEM((CB, D), jnp.float32)] * NBUF,
            [pltpu.SemaphoreType.DMA] * NBUF,
            [pltpu.SemaphoreType.DMA] * NBUF,
        ],
    )
    def kern(idx_hbm, table_hbm, out_hbm, idx_v, rows, gsem, osem):
        wid = lax.axis_index("s") * NC + lax.axis_index("c")
        wb = wid * CB
        pltpu.sync_copy(idx_hbm.at[:, pl.ds(wb, CB)], idx_v)

        def body(k, carry):
            for b in range(NBUF):
                @pl.when(((k % NBUF) == b) & (k < K))
                def _():
                    @pl.when(k >= NBUF)
                    def _():
                        pltpu.make_async_copy(
                            out_hbm.at[k - NBUF, pl.ds(wb, CB)], rows[b],
                            osem[b]).wait()
                    sp = CB // NSPL
                    for j in range(NSPL):
                        pltpu.async_copy(
                            table_hbm.at[idx_v.at[k, pl.ds(j * sp, sp)]],
                            rows[b].at[pl.ds(j * sp, sp)], gsem[b])
            for b in range(NBUF):
                @pl.when((((k - 1) % NBUF) == b) & (k >= 1) & (k <= K))
                def _():
                    pltpu.make_async_copy(
                        out_hbm.at[k - 1, pl.ds(wb, CB)], rows[b],
                        gsem[b]).wait()
                    pltpu.async_copy(rows[b],
                                     out_hbm.at[k - 1, pl.ds(wb, CB)],
                                     osem[b])
            return carry

        lax.fori_loop(0, K + 1, body, 0, unroll=False)

        for b in range(NBUF):
            pltpu.make_async_copy(out_hbm.at[0, pl.ds(wb, CB)], rows[b],
                                  osem[b]).wait()

    return kern(idx, table)


def kernel(fiber_idx, s):
    out_km = _sc_gather(fiber_idx.astype(jnp.int32).T, s)
    return out_km.transpose(1, 0, 2)


# speedup vs baseline: 1.0056x; 1.0056x over previous
import functools

import jax
import jax.numpy as jnp
from jax import lax
from jax.experimental import pallas as pl
from jax.experimental.pallas import tpu as pltpu
from jax.experimental.pallas import tpu_sc as plsc

NC = 2
NS = 16
NW = NC * NS
NBUF = 4
NSPL = 2


@jax.jit
def _sc_gather(idx, table):
    K, B = idx.shape
    _, D = table.shape
    CB = B // NW
    mesh = plsc.VectorSubcoreMesh(core_axis_name="c", subcore_axis_name="s")

    @functools.partial(
        pl.kernel,
        out_type=jax.ShapeDtypeStruct((K, B, D), jnp.float32),
        mesh=mesh,
        scratch_types=[
            pltpu.VMEM((K, CB), jnp.int32),
            [pltpu.VMEM((CB, D), jnp.float32)] * NBUF,
            [pltpu.SemaphoreType.DMA] * NBUF,
            [pltpu.SemaphoreType.DMA] * NBUF,
        ],
    )
    def kern(idx_hbm, table_hbm, out_hbm, idx_v, rows, gsem, osem):
        wid = lax.axis_index("s") * NC + lax.axis_index("c")
        wb = wid * CB
        pltpu.sync_copy(idx_hbm.at[:, pl.ds(wb, CB)], idx_v)

        def body(k, carry):
            for b in range(NBUF):
                @pl.when(((k % NBUF) == b) & (k < K))
                def _():
                    @pl.when(k >= NBUF)
                    def _():
                        pltpu.make_async_copy(
                            out_hbm.at[k - NBUF, pl.ds(wb, CB)], rows[b],
                            osem[b]).wait()
                    sp = CB // NSPL
                    for j in range(NSPL):
                        pltpu.async_copy(
                            table_hbm.at[idx_v.at[k, pl.ds(j * sp, sp)]],
                            rows[b].at[pl.ds(j * sp, sp)], gsem[b])
            for b in range(NBUF):
                @pl.when((((k - 1) % NBUF) == b) & (k >= 1) & (k <= K))
                def _():
                    pltpu.make_async_copy(
                        out_hbm.at[k - 1, pl.ds(wb, CB)], rows[b],
                        gsem[b]).wait()
                    pltpu.async_copy(rows[b],
                                     out_hbm.at[k - 1, pl.ds(wb, CB)],
                                     osem[b])
            return carry

        lax.fori_loop(0, K + 1, body, 0, unroll=False)

        for b in range(NBUF):
            pltpu.make_async_copy(out_hbm.at[0, pl.ds(wb, CB)], rows[b],
                                  osem[b]).wait()

    return kern(idx, table)


def kernel(fiber_idx, s):
    out_km = _sc_gather(fiber_idx.astype(jnp.int32).T, s)
    return out_km.transpose(1, 0, 2)
